# baseline (speedup 1.0000x reference)
import functools

import jax
import jax.numpy as jnp
from jax.experimental import pallas as pl
from jax.experimental.pallas import tpu as pltpu


_TAU = 0.9
_P = 0.8
_MIN_K = 1
_E = 16
_BB = 1024
_HC = 8
_NJ = 64 // _HC


def _router_block(xe_ref, xo_ref, web_ref, wob_ref, convb_ref, fcw_ref,
                  fcb_ref, out_ref, pooled_ref):
    j = pl.program_id(1)
    xet = jnp.transpose(xe_ref[...], (1, 0, 2))
    xot = jnp.transpose(xo_ref[...], (1, 0, 2))
    y = jax.lax.dot_general(
        web_ref[...], xet, (((2,), (1,)), ((0,), (0,))),
        preferred_element_type=jnp.float32)
    y = y + jax.lax.dot_general(
        wob_ref[...], xot, (((2,), (1,)), ((0,), (0,))),
        preferred_element_type=jnp.float32)
    z = jnp.maximum(y + convb_ref[...][None, :, :], 0.0)
    contrib = jnp.sum(z, axis=0)

    @pl.when(j == 0)
    def _():
        pooled_ref[...] = contrib

    @pl.when(j > 0)
    def _():
        pooled_ref[...] = pooled_ref[...] + contrib

    @pl.when(j == _NJ - 1)
    def _():
        logits = jax.lax.dot_general(
            fcw_ref[...], pooled_ref[...], (((1,), (0,)), ((), ())),
            preferred_element_type=jnp.float32) + fcb_ref[...]
        zl = logits * (1.0 / _TAU)
        zl = zl - jnp.max(zl, axis=0, keepdims=True)
        e = jnp.exp(zl)
        probs = e / jnp.sum(e, axis=0, keepdims=True)

        cums = jnp.zeros_like(probs)
        rank = jnp.zeros_like(probs)
        i_idx = jax.lax.broadcasted_iota(jnp.int32, (_E, 1), 0)
        for k in range(_E):
            vk = probs[k:k + 1, :]
            prec_incl = (vk > probs) | ((vk == probs) & (k <= i_idx))
            cums = cums + jnp.where(prec_incl, vk, 0.0)
            rank = rank + jnp.where(prec_incl, 1.0, 0.0)
        keep = (cums <= _P) | (rank - 1.0 < _MIN_K)
        masked = jnp.where(keep, probs, 0.0)
        denom = jnp.clip(jnp.sum(masked, axis=0, keepdims=True), 1e-10, None)
        out_ref[...] = masked / denom


@functools.partial(jax.jit, static_argnames=())
def _run(patch, conv_w, conv_b, fc_w, fc_b):
    B = patch.shape[0]
    x3 = jnp.transpose(patch.reshape(B, 98, 128), (1, 2, 0))
    we = conv_w[:, 0::2]
    wo = conv_w[:, 1::2]
    web = jnp.broadcast_to(we[None], (_HC, 128, 98))
    wob = jnp.broadcast_to(wo[None], (_HC, 128, 98))
    conv_b2 = conv_b.reshape(128, 1)
    fcw_s = fc_w * (1.0 / 64.0)
    fc_b2 = fc_b.reshape(_E, 1)
    out_t = pl.pallas_call(
        _router_block,
        grid=(B // _BB, _NJ),
        in_specs=[
            pl.BlockSpec((98, _HC, _BB), lambda i, j: (0, j, i)),
            pl.BlockSpec((98, _HC, _BB), lambda i, j: (0, _NJ + j, i)),
            pl.BlockSpec((_HC, 128, 98), lambda i, j: (0, 0, 0)),
            pl.BlockSpec((_HC, 128, 98), lambda i, j: (0, 0, 0)),
            pl.BlockSpec((128, 1), lambda i, j: (0, 0)),
            pl.BlockSpec((_E, 128), lambda i, j: (0, 0)),
            pl.BlockSpec((_E, 1), lambda i, j: (0, 0)),
        ],
        out_specs=pl.BlockSpec((_E, _BB), lambda i, j: (0, i)),
        out_shape=jax.ShapeDtypeStruct((_E, B), jnp.float32),
        scratch_shapes=[pltpu.VMEM((128, _BB), jnp.float32)],
    )(x3, x3, web, wob, conv_b2, fcw_s, fc_b2)
    return out_t.T


def kernel(patch, conv_w, conv_b, fc_w, fc_b, layer_idx, threshold):
    return _run(patch, conv_w, conv_b, fc_w, fc_b)


# speedup vs baseline: 1.0459x; 1.0459x over previous
import functools

import jax
import jax.numpy as jnp
from jax.experimental import pallas as pl
from jax.experimental.pallas import tpu as pltpu


_TAU = 0.9
_P = 0.8
_MIN_K = 1
_E = 16
_BB = 1024
_HC = 8
_NJ = 64 // _HC


def _router_block(xe_ref, xo_ref, web_ref, convb_ref, fcw_ref,
                  fcb_ref, out_ref, pooled_ref):
    j = pl.program_id(1)
    xet = jnp.transpose(xe_ref[...], (1, 0, 2))
    xot = jnp.transpose(xo_ref[...], (1, 0, 2))
    xcat = jnp.concatenate((xet, xot), axis=1)
    y = jax.lax.dot_general(
        web_ref[...], xcat, (((2,), (1,)), ((0,), (0,))),
        preferred_element_type=jnp.float32)
    z = jnp.maximum(y + convb_ref[...][None, :, :], 0.0)
    contrib = jnp.sum(z, axis=0)

    @pl.when(j == 0)
    def _():
        pooled_ref[...] = contrib

    @pl.when(j > 0)
    def _():
        pooled_ref[...] = pooled_ref[...] + contrib

    @pl.when(j == _NJ - 1)
    def _():
        logits = jax.lax.dot_general(
            fcw_ref[...], pooled_ref[...], (((1,), (0,)), ((), ())),
            preferred_element_type=jnp.float32) + fcb_ref[...]
        zl = logits * (1.0 / _TAU)
        zl = zl - jnp.max(zl, axis=0, keepdims=True)
        e = jnp.exp(zl)
        probs = e / jnp.sum(e, axis=0, keepdims=True)

        cums = jnp.zeros_like(probs)
        rank = jnp.zeros_like(probs)
        i_idx = jax.lax.broadcasted_iota(jnp.int32, (_E, 1), 0)
        for k in range(_E):
            vk = probs[k:k + 1, :]
            prec_incl = (vk > probs) | ((vk == probs) & (k <= i_idx))
            cums = cums + jnp.where(prec_incl, vk, 0.0)
            rank = rank + jnp.where(prec_incl, 1.0, 0.0)
        keep = (cums <= _P) | (rank - 1.0 < _MIN_K)
        masked = jnp.where(keep, probs, 0.0)
        denom = jnp.clip(jnp.sum(masked, axis=0, keepdims=True), 1e-10, None)
        out_ref[...] = masked / denom


@functools.partial(jax.jit, static_argnames=())
def _run(patch, conv_w, conv_b, fc_w, fc_b):
    B = patch.shape[0]
    x3 = jnp.transpose(patch.reshape(B, 98, 128), (1, 2, 0))
    wcat = jnp.concatenate((conv_w[:, 0::2], conv_w[:, 1::2]), axis=1)
    web = jnp.broadcast_to(wcat[None], (_HC, 128, 196))
    conv_b2 = conv_b.reshape(128, 1)
    fcw_s = fc_w * (1.0 / 64.0)
    fc_b2 = fc_b.reshape(_E, 1)
    out_t = pl.pallas_call(
        _router_block,
        grid=(B // _BB, _NJ),
        in_specs=[
            pl.BlockSpec((98, _HC, _BB), lambda i, j: (0, j, i)),
            pl.BlockSpec((98, _HC, _BB), lambda i, j: (0, _NJ + j, i)),
            pl.BlockSpec((_HC, 128, 196), lambda i, j: (0, 0, 0)),
            pl.BlockSpec((128, 1), lambda i, j: (0, 0)),
            pl.BlockSpec((_E, 128), lambda i, j: (0, 0)),
            pl.BlockSpec((_E, 1), lambda i, j: (0, 0)),
        ],
        out_specs=pl.BlockSpec((_E, _BB), lambda i, j: (0, i)),
        out_shape=jax.ShapeDtypeStruct((_E, B), jnp.float32),
        scratch_shapes=[pltpu.VMEM((128, _BB), jnp.float32)],
    )(x3, x3, web, conv_b2, fcw_s, fc_b2)
    return out_t.T


def kernel(patch, conv_w, conv_b, fc_w, fc_b, layer_idx, threshold):
    return _run(patch, conv_w, conv_b, fc_w, fc_b)
